# Initial kernel scaffold; baseline (speedup 1.0000x reference)
import jax
import jax.numpy as jnp
from jax.experimental import pallas as pl


def kernel(router_logits):
    raise NotImplementedError("write your pallas kernel here")


# speedup vs baseline: 1.5275x; 1.5275x over previous
import functools

import jax
import jax.numpy as jnp
from jax import lax
from jax.experimental import pallas as pl
from jax.experimental.pallas import tpu as pltpu
from jax.experimental.pallas import tpu_sc as plsc

_E = 64
_G = 4
_T = 8192
_NW = 32
_TPW = (_G * _T) // _NW
_OUTW = 256

_LN2 = 0.6931471805599453
_SQRT2 = 1.4142135623730951


def _one_token(in_v, o, idx0, idx1, idx2, idx3, big, ninf, zero,
               ap0, ap1, ap2, ap3, ac0, ac1, ac2, ac3):
    a = in_v[pl.ds(o, 16)]
    b = in_v[pl.ds(o + 16, 16)]
    c = in_v[pl.ds(o + 32, 16)]
    d = in_v[pl.ds(o + 48, 16)]
    m1 = jnp.max(jnp.maximum(jnp.maximum(a, b), jnp.maximum(c, d)))
    u0 = jnp.exp(a - m1)
    u1 = jnp.exp(b - m1)
    u2 = jnp.exp(c - m1)
    u3 = jnp.exp(d - m1)
    s = jnp.sum((u0 + u1) + (u2 + u3))
    c0 = jnp.where(a == m1, idx0, big)
    c1 = jnp.where(b == m1, idx1, big)
    c2 = jnp.where(c == m1, idx2, big)
    c3 = jnp.where(d == m1, idx3, big)
    i1 = jnp.min(jnp.minimum(jnp.minimum(c0, c1), jnp.minimum(c2, c3)))
    xa = jnp.where(idx0 == i1, ninf, a)
    xb = jnp.where(idx1 == i1, ninf, b)
    xc = jnp.where(idx2 == i1, ninf, c)
    xd = jnp.where(idx3 == i1, ninf, d)
    m2 = jnp.max(jnp.maximum(jnp.maximum(xa, xb), jnp.maximum(xc, xd)))
    e0 = jnp.where(xa == m2, idx0, big)
    e1 = jnp.where(xb == m2, idx1, big)
    e2 = jnp.where(xc == m2, idx2, big)
    e3 = jnp.where(xd == m2, idx3, big)
    i2 = jnp.min(jnp.minimum(jnp.minimum(e0, e1), jnp.minimum(e2, e3)))
    one16 = jnp.full((16,), 1.0, jnp.float32)
    sinv = one16 / s
    ap0 = ap0 + u0 * sinv
    ap1 = ap1 + u1 * sinv
    ap2 = ap2 + u2 * sinv
    ap3 = ap3 + u3 * sinv
    ac0 = ac0 + jnp.where((idx0 == i1) | (idx0 == i2), one16, zero)
    ac1 = ac1 + jnp.where((idx1 == i1) | (idx1 == i2), one16, zero)
    ac2 = ac2 + jnp.where((idx2 == i1) | (idx2 == i2), one16, zero)
    ac3 = ac3 + jnp.where((idx3 == i1) | (idx3 == i2), one16, zero)
    return ap0, ap1, ap2, ap3, ac0, ac1, ac2, ac3, m1, s


def _sc_body(x_hbm, out_hbm, in_v, out_v):
    cid = lax.axis_index("c")
    sid = lax.axis_index("s")
    wid = sid * 2 + cid
    base = wid * (_TPW * _E)

    pltpu.sync_copy(x_hbm.at[pl.ds(base, _TPW * _E)], in_v)

    iota = lax.iota(jnp.int32, 16)
    idx0 = iota
    idx1 = iota + 16
    idx2 = iota + 32
    idx3 = iota + 48
    zero = jnp.zeros((16,), jnp.float32)
    big = jnp.int32(_E)
    ninf = jnp.float32(-jnp.inf)

    def batch_body(t, carry):
        ap0, ap1, ap2, ap3, ac0, ac1, ac2, ac3, zacc = carry
        mvec = zero
        svec = zero
        for k in range(16):
            o = t * (16 * _E) + k * _E
            ap0, ap1, ap2, ap3, ac0, ac1, ac2, ac3, m1, s = _one_token(
                in_v, o, idx0, idx1, idx2, idx3, big, ninf, zero,
                ap0, ap1, ap2, ap3, ac0, ac1, ac2, ac3)
            mvec = jnp.where(iota == k, m1, mvec)
            svec = jnp.where(iota == k, s, svec)
        bits = lax.bitcast_convert_type(svec, jnp.int32)
        ex = lax.shift_right_logical(bits, 23) - 127
        mant = lax.bitcast_convert_type(
            (bits & 0x7FFFFF) | 0x3F800000, jnp.float32)
        hi = mant > _SQRT2
        mant = jnp.where(hi, mant * 0.5, mant)
        exf = ex.astype(jnp.float32) + jnp.where(hi, 1.0, 0.0)
        r = (mant - 1.0) / (mant + 1.0)
        r2 = r * r
        lnm = 2.0 * r * (1.0 + r2 * (1.0 / 3.0 + r2 * (0.2 + r2 * (1.0 / 7.0))))
        lz = mvec + (exf * _LN2 + lnm)
        zacc = zacc + lz * lz
        return (ap0, ap1, ap2, ap3, ac0, ac1, ac2, ac3, zacc)

    init = (zero,) * 9
    ap0, ap1, ap2, ap3, ac0, ac1, ac2, ac3, zvec = lax.fori_loop(
        0, _TPW // 16, batch_body, init)
    zsq = jnp.sum(zvec)

    out_v[pl.ds(0, 16)] = ap0
    out_v[pl.ds(16, 16)] = ap1
    out_v[pl.ds(32, 16)] = ap2
    out_v[pl.ds(48, 16)] = ap3
    out_v[pl.ds(64, 16)] = ac0
    out_v[pl.ds(80, 16)] = ac1
    out_v[pl.ds(96, 16)] = ac2
    out_v[pl.ds(112, 16)] = ac3
    out_v[pl.ds(128, 16)] = jnp.where(iota == 0, zsq, 0.0)
    for k in range(9, 16):
        out_v[pl.ds(k * 16, 16)] = zero
    pltpu.sync_copy(out_v, out_hbm.at[pl.ds(wid * _OUTW, _OUTW)])


_sc_router = functools.partial(
    pl.kernel,
    out_type=jax.ShapeDtypeStruct((_NW * _OUTW,), jnp.float32),
    mesh=plsc.VectorSubcoreMesh(core_axis_name="c", subcore_axis_name="s"),
    compiler_params=pltpu.CompilerParams(needs_layout_passes=False),
    scratch_types=[
        pltpu.VMEM((_TPW * _E,), jnp.float32),
        pltpu.VMEM((_OUTW,), jnp.float32),
    ],
)(_sc_body)


def _combine_body(b_ref, o_ref):
    b = b_ref[...]
    aux = jnp.float32(0.0)
    for g in range(_G):
        rows = b[g * 8:(g + 1) * 8, :]
        p = jnp.sum(rows[:, 0:_E], axis=0, keepdims=True)
        c = jnp.sum(rows[:, _E:2 * _E], axis=0, keepdims=True)
        aux = aux + jnp.sum(p * c)
    aux = aux * jnp.float32(
        float(_E) * _E / (float(_G) * _E * float(_T) * _T))
    z = jnp.sum(b[:, 2 * _E:]) / jnp.float32(_G * _T)
    row = lax.broadcasted_iota(jnp.int32, (8, 128), 0)
    lane = lax.broadcasted_iota(jnp.int32, (8, 128), 1)
    o = jnp.where((row == 0) & (lane == 0), aux, 0.0)
    o = jnp.where((row == 0) & (lane == 1), z, o)
    o_ref[...] = o


def _combine(buf):
    return pl.pallas_call(
        _combine_body,
        out_shape=jax.ShapeDtypeStruct((8, 128), jnp.float32),
    )(buf)


def kernel(router_logits):
    x = router_logits.reshape(-1)
    partials = _sc_router(x)
    out = _combine(partials.reshape(_NW, _OUTW))
    return out[0, 0:2]
